# baseline (speedup 1.0000x reference)
import functools

import jax
import jax.numpy as jnp
from jax import lax
from jax.experimental import pallas as pl
from jax.experimental.pallas import tpu as pltpu
from jax.experimental.pallas import tpu_sc as plsc

N = 10000
E = 320000
G = 64
DIN = 768
DH = 128

NC = 2
NS = 16
NW = NC * NS
LANES = 16

NP = 10240
EP = 327680
NG = 80
HG = 40
ROWS_PER_TILE = NP // NW
ROWS_PER_SUB = NP // NS

_mesh = functools.partial(
    plsc.VectorSubcoreMesh, core_axis_name="c", subcore_axis_name="s",
    num_cores=NC, num_subcores=NS)


def _rsqrt16(d):
    i = lax.bitcast_convert_type(d, jnp.int32)
    i = jnp.int32(0x5F3759DF) - lax.shift_right_arithmetic(i, 1)
    y = lax.bitcast_convert_type(i, jnp.float32)
    for _ in range(3):
        y = y * (jnp.float32(1.5) - jnp.float32(0.5) * d * y * y)
    return y


def _deg_s_body(sd_ref, s_ref, sd_v, ones_v, zeros_v, sv, s_blk, hist_sh, semh):
    c = lax.axis_index("c")
    sid = lax.axis_index("s")

    def _fill(i, _):
        zeros_v[pl.ds(i * LANES, LANES)] = jnp.zeros((LANES,), jnp.float32)
        return 0
    lax.fori_loop(0, ROWS_PER_SUB // LANES, _fill, 0)
    for k in range(8):
        ones_v[pl.ds(k * LANES, LANES)] = jnp.ones((LANES,), jnp.float32)

    pltpu.sync_copy(zeros_v, hist_sh.at[pl.ds(sid * ROWS_PER_SUB, ROWS_PER_SUB)])
    plsc.subcore_barrier()

    def _chunk(chunk_idx):
        pltpu.sync_copy(sd_ref.at[pl.ds(chunk_idx * NG, NG)], sd_v)
        for j in range(NG):
            pltpu.async_copy(ones_v, hist_sh.at[sd_v.at[j, 1]], semh, add=True)
        for j in range(NG):
            pltpu.make_async_copy(ones_v, hist_sh.at[sd_v.at[j, 1]], semh).wait()

    _chunk(sid)
    _chunk(sid + NS)
    plsc.subcore_barrier()

    base = c * (NP // NC) + sid * ROWS_PER_TILE
    pltpu.sync_copy(hist_sh.at[pl.ds(base, ROWS_PER_TILE)], sv)

    def _scal(i, _):
        d = sv[pl.ds(i * LANES, LANES)] + jnp.float32(1.0)
        sv[pl.ds(i * LANES, LANES)] = _rsqrt16(d)
        return 0
    lax.fori_loop(0, ROWS_PER_TILE // LANES, _scal, 0)

    def _bcast(c16, _):
        v = sv[pl.ds(c16 * LANES, LANES)]
        for j in range(LANES):
            row = c16 * LANES + j
            vj = jnp.full((LANES,), v[j], jnp.float32)
            for k in range(DH // LANES):
                s_blk[row, pl.ds(k * LANES, LANES)] = vj
        return 0
    lax.fori_loop(0, ROWS_PER_TILE // LANES, _bcast, 0)

    pltpu.sync_copy(s_blk, s_ref.at[pl.ds(base, ROWS_PER_TILE)])


def _deg_s(sd):
    return pl.kernel(
        _deg_s_body,
        out_type=jax.ShapeDtypeStruct((NP, DH), jnp.float32),
        mesh=_mesh(),
        scratch_types=[
            pltpu.VMEM((NG, 2, 128), jnp.int32),
            pltpu.VMEM((128,), jnp.float32),
            pltpu.VMEM((ROWS_PER_SUB,), jnp.float32),
            pltpu.VMEM((ROWS_PER_TILE,), jnp.float32),
            pltpu.VMEM((ROWS_PER_TILE, DH), jnp.float32),
            pltpu.VMEM_SHARED((NP,), jnp.float32),
            pltpu.SemaphoreType.DMA,
        ],
    )(sd)


def _scatter_body(g_ref, sd_ref, acc_ref,
                  sd_v, rows, sem0, sem1, sems0, sems1, acc_sh):
    c = lax.axis_index("c")
    sid = lax.axis_index("s")
    wid = c * NS + sid

    def _z(r, _):
        for k in range(DH // LANES):
            rows[0, r, pl.ds(k * LANES, LANES)] = jnp.zeros((LANES,), jnp.float32)
        return 0
    lax.fori_loop(0, 128, _z, 0)
    for k in range(ROWS_PER_SUB // 128):
        pltpu.sync_copy(rows.at[0],
                        acc_sh.at[pl.ds(sid * ROWS_PER_SUB + k * 128, 128)])
    plsc.subcore_barrier()

    def _gat(j, b):
        return pltpu.make_async_copy(g_ref.at[sd_v.at[j, 0]], rows.at[b], sem0 if b == 0 else sem1)

    def _sca(j, b):
        return pltpu.make_async_copy(rows.at[b], acc_sh.at[sd_v.at[j, 1]], sems0 if b == 0 else sems1)

    for h in range(NG // HG):
        pltpu.sync_copy(sd_ref.at[pl.ds(wid * NG + h * HG, HG)], sd_v)
        _gat(0, 0).start()
        _gat(1, 1).start()

        def _grp(i, _):
            j = i * 2
            _gat(j, 0).wait()
            pltpu.async_copy(rows.at[0], acc_sh.at[sd_v.at[j, 1]], sems0, add=True)
            _gat(j + 1, 1).wait()
            pltpu.async_copy(rows.at[1], acc_sh.at[sd_v.at[j + 1, 1]], sems1, add=True)

            @pl.when(j + 2 < HG)
            def _():
                _sca(j, 0).wait()
                _gat(j + 2, 0).start()

            @pl.when(j + 3 < HG)
            def _():
                _sca(j + 1, 1).wait()
                _gat(j + 3, 1).start()
            return 0
        lax.fori_loop(0, HG // 2, _grp, 0)
        _sca(HG - 2, 0).wait()
        _sca(HG - 1, 1).wait()
    plsc.subcore_barrier()

    for k in range(ROWS_PER_SUB // 128):
        base = sid * ROWS_PER_SUB + k * 128
        pltpu.sync_copy(acc_sh.at[pl.ds(base, 128)], rows.at[0])
        pltpu.sync_copy(rows.at[0], acc_ref.at[c, pl.ds(base, 128)])


def _scatter(g, sd):
    return pl.kernel(
        _scatter_body,
        out_type=jax.ShapeDtypeStruct((NC, NP, DH), jnp.float32),
        mesh=_mesh(),
        scratch_types=[
            pltpu.VMEM((HG, 2, 128), jnp.int32),
            pltpu.VMEM((2, 128, DH), jnp.float32),
            pltpu.SemaphoreType.DMA,
            pltpu.SemaphoreType.DMA,
            pltpu.SemaphoreType.DMA,
            pltpu.SemaphoreType.DMA,
            pltpu.VMEM_SHARED((NP, DH), jnp.float32),
        ],
    )(g, sd)


RB = 1000
RB3 = 1024


def _mm_body(x_ref, w_ref, o_ref):
    o_ref[...] = jnp.dot(x_ref[...], w_ref[...],
                         preferred_element_type=jnp.float32)


def _matmul(x, W1):
    return pl.pallas_call(
        _mm_body,
        grid=(N // RB,),
        in_specs=[pl.BlockSpec((RB, DIN), lambda i: (i, 0)),
                  pl.BlockSpec((DIN, DH), lambda i: (0, 0))],
        out_specs=pl.BlockSpec((RB, DH), lambda i: (i, 0)),
        out_shape=jax.ShapeDtypeStruct((N, DH), jnp.float32),
    )(x, W1)


def _scale_body(h_ref, s_ref, o_ref):
    o_ref[...] = h_ref[...] * s_ref[...]


def _scale(h, s_b):
    return pl.pallas_call(
        _scale_body,
        grid=(N // RB,),
        in_specs=[pl.BlockSpec((RB, DH), lambda i: (i, 0)),
                  pl.BlockSpec((RB, DH), lambda i: (i, 0))],
        out_specs=pl.BlockSpec((RB, DH), lambda i: (i, 0)),
        out_shape=jax.ShapeDtypeStruct((N, DH), jnp.float32),
    )(h, s_b)


def _layer2_body(a_ref, g1_ref, s_ref, b1_ref, w2_ref, o_ref):
    s = s_ref[...]
    t = (a_ref[0] + a_ref[1] + g1_ref[...]) * s + b1_ref[...]
    t = jnp.maximum(t, jnp.float32(0.0))
    o_ref[...] = jnp.dot(t, w2_ref[...], preferred_element_type=jnp.float32) * s


def _layer2(acc, g1, s_b, b1r, W2):
    blk = pl.BlockSpec((RB, DH), lambda i: (i, 0))
    return pl.pallas_call(
        _layer2_body,
        grid=(N // RB,),
        in_specs=[pl.BlockSpec((NC, RB, DH), lambda i: (0, i, 0)),
                  blk, blk,
                  pl.BlockSpec((1, DH), lambda i: (0, 0)),
                  pl.BlockSpec((DH, DH), lambda i: (0, 0))],
        out_specs=blk,
        out_shape=jax.ShapeDtypeStruct((N, DH), jnp.float32),
    )(acc, g1, s_b, b1r, W2)


def _pool_body(a_ref, g2_ref, s_ref, b2_ref, batch_ref, o_ref, sums, cnts):
    i = pl.program_id(0)

    @pl.when(i == 0)
    def _():
        sums[...] = jnp.zeros((G, DH), jnp.float32)
        cnts[...] = jnp.zeros((G, DH), jnp.float32)

    h2 = (a_ref[0] + a_ref[1] + g2_ref[...]) * s_ref[...] + b2_ref[...]
    bblk = batch_ref[0]
    gids = lax.broadcasted_iota(jnp.int32, (G, 128), 0)
    for r4 in range(RB3 // 128):
        brow = bblk[r4:r4 + 1, :]
        mask = (brow == gids).astype(jnp.float32)
        sums[...] += jnp.dot(mask, h2[r4 * 128:(r4 + 1) * 128, :],
                             preferred_element_type=jnp.float32)
        cnts[...] += jnp.sum(mask, axis=1, keepdims=True)

    @pl.when(i == NP // RB3 - 1)
    def _():
        o_ref[...] = sums[...] / jnp.maximum(cnts[...], jnp.float32(1.0))


def _pool(acc, g2p, s_b, b2r, batch_p):
    blk = pl.BlockSpec((RB3, DH), lambda i: (i, 0))
    return pl.pallas_call(
        _pool_body,
        grid=(NP // RB3,),
        in_specs=[pl.BlockSpec((NC, RB3, DH), lambda i: (0, i, 0)),
                  blk, blk,
                  pl.BlockSpec((1, DH), lambda i: (0, 0)),
                  pl.BlockSpec((1, RB3 // 128, 128), lambda i: (i, 0, 0))],
        out_specs=pl.BlockSpec((G, DH), lambda i: (0, 0)),
        out_shape=jax.ShapeDtypeStruct((G, DH), jnp.float32),
        scratch_shapes=[pltpu.VMEM((G, DH), jnp.float32),
                        pltpu.VMEM((G, DH), jnp.float32)],
    )(acc, g2p, s_b, b2r, batch_p)


def kernel(x, edge_index, batch, W1, b1, W2, b2):
    npad = EP - E
    pad_src = jnp.arange(npad, dtype=jnp.int32) % N
    pad_dst = (jnp.arange(npad, dtype=jnp.int32) % (NP - N)) + N
    sd = jnp.concatenate(
        [edge_index.astype(jnp.int32), jnp.stack([pad_src, pad_dst])],
        axis=1).reshape(2, EP // 128, 128).transpose(1, 0, 2)
    batch_p = jnp.concatenate(
        [batch.astype(jnp.int32),
         jnp.full((NP - N,), 127, jnp.int32)]).reshape(NP // RB3, RB3 // 128, 128)
    b1r = b1.reshape(1, DH)
    b2r = b2.reshape(1, DH)

    h1 = _matmul(x, W1)
    s_b = _deg_s(sd)
    g1 = _scale(h1, s_b)
    acc1 = _scatter(g1, sd)
    g2 = _layer2(acc1, g1, s_b, b1r, W2)
    acc2 = _scatter(g2, sd)
    g2p = jnp.pad(g2, ((0, NP - N), (0, 0)))
    return _pool(acc2, g2p, s_b, b2r, batch_p)


# speedup vs baseline: 1.2484x; 1.2484x over previous
import functools

import jax
import jax.numpy as jnp
from jax import lax
from jax.experimental import pallas as pl
from jax.experimental.pallas import tpu as pltpu
from jax.experimental.pallas import tpu_sc as plsc

N = 10000
E = 320000
G = 64
DIN = 768
DH = 128

NC = 2
NS = 16
NW = NC * NS
LANES = 16

NP = 10240
EP = 327680
NG = 80
HG = 40
ROWS_PER_TILE = NP // NW
ROWS_PER_SUB = NP // NS

_mesh = functools.partial(
    plsc.VectorSubcoreMesh, core_axis_name="c", subcore_axis_name="s",
    num_cores=NC, num_subcores=NS)


def _rsqrt16(d):
    i = lax.bitcast_convert_type(d, jnp.int32)
    i = jnp.int32(0x5F3759DF) - lax.shift_right_arithmetic(i, 1)
    y = lax.bitcast_convert_type(i, jnp.float32)
    for _ in range(3):
        y = y * (jnp.float32(1.5) - jnp.float32(0.5) * d * y * y)
    return y


def _deg_s_body(sd_ref, s_ref, sd_v, ones_v, zeros_v, sv, s_blk, hist_sh, semh):
    c = lax.axis_index("c")
    sid = lax.axis_index("s")

    def _fill(i, _):
        zeros_v[pl.ds(i * LANES, LANES)] = jnp.zeros((LANES,), jnp.float32)
        return 0
    lax.fori_loop(0, ROWS_PER_SUB // LANES, _fill, 0)
    for k in range(8):
        ones_v[pl.ds(k * LANES, LANES)] = jnp.ones((LANES,), jnp.float32)

    pltpu.sync_copy(zeros_v, hist_sh.at[pl.ds(sid * ROWS_PER_SUB, ROWS_PER_SUB)])
    plsc.subcore_barrier()

    def _chunk(chunk_idx):
        pltpu.sync_copy(sd_ref.at[pl.ds(chunk_idx * NG, NG)], sd_v)
        for j in range(NG):
            pltpu.async_copy(ones_v, hist_sh.at[sd_v.at[j, 1]], semh, add=True)
        for j in range(NG):
            pltpu.make_async_copy(ones_v, hist_sh.at[sd_v.at[j, 1]], semh).wait()

    _chunk(sid)
    _chunk(sid + NS)
    plsc.subcore_barrier()

    base = c * (NP // NC) + sid * ROWS_PER_TILE
    pltpu.sync_copy(hist_sh.at[pl.ds(base, ROWS_PER_TILE)], sv)

    def _scal(i, _):
        d = sv[pl.ds(i * LANES, LANES)] + jnp.float32(1.0)
        sv[pl.ds(i * LANES, LANES)] = _rsqrt16(d)
        return 0
    lax.fori_loop(0, ROWS_PER_TILE // LANES, _scal, 0)

    def _bcast(c16, _):
        v = sv[pl.ds(c16 * LANES, LANES)]
        for j in range(LANES):
            row = c16 * LANES + j
            vj = jnp.full((LANES,), v[j], jnp.float32)
            for k in range(DH // LANES):
                s_blk[row, pl.ds(k * LANES, LANES)] = vj
        return 0
    lax.fori_loop(0, ROWS_PER_TILE // LANES, _bcast, 0)

    pltpu.sync_copy(s_blk, s_ref.at[pl.ds(base, ROWS_PER_TILE)])


def _deg_s(sd):
    return pl.kernel(
        _deg_s_body,
        out_type=jax.ShapeDtypeStruct((NP, DH), jnp.float32),
        mesh=_mesh(),
        scratch_types=[
            pltpu.VMEM((NG, 2, 128), jnp.int32),
            pltpu.VMEM((128,), jnp.float32),
            pltpu.VMEM((ROWS_PER_SUB,), jnp.float32),
            pltpu.VMEM((ROWS_PER_TILE,), jnp.float32),
            pltpu.VMEM((ROWS_PER_TILE, DH), jnp.float32),
            pltpu.VMEM_SHARED((NP,), jnp.float32),
            pltpu.SemaphoreType.DMA,
        ],
    )(sd)


def _scatter_body(g_ref, sd_ref, acc_ref,
                  sd_v, rows, sem0, sem1, acc_sh):
    c = lax.axis_index("c")
    sid = lax.axis_index("s")
    wid = c * NS + sid

    def _z(r, _):
        for k in range(DH // LANES):
            rows[0, r, pl.ds(k * LANES, LANES)] = jnp.zeros((LANES,), jnp.float32)
        return 0
    lax.fori_loop(0, 128, _z, 0)
    for k in range(ROWS_PER_SUB // 128):
        pltpu.sync_copy(rows.at[0],
                        acc_sh.at[pl.ds(sid * ROWS_PER_SUB + k * 128, 128)])
    plsc.subcore_barrier()

    def _gat(j, b):
        return pltpu.make_async_copy(g_ref.at[sd_v.at[j, 0]], rows.at[b], sem0 if b == 0 else sem1)

    for h in range(NG // HG):
        pltpu.sync_copy(sd_ref.at[pl.ds(wid * NG + h * HG, HG)], sd_v)
        _gat(0, 0).start()

        def _grp(i, _):
            j = i * 2
            _gat(j + 1, 1).start()
            _gat(j, 0).wait()
            pltpu.sync_copy(rows.at[0], acc_sh.at[sd_v.at[j, 1]], add=True)

            @pl.when(j + 2 < HG)
            def _():
                _gat(j + 2, 0).start()
            _gat(j + 1, 1).wait()
            pltpu.sync_copy(rows.at[1], acc_sh.at[sd_v.at[j + 1, 1]], add=True)
            return 0
        lax.fori_loop(0, HG // 2, _grp, 0)
    plsc.subcore_barrier()

    for k in range(ROWS_PER_SUB // 128):
        base = sid * ROWS_PER_SUB + k * 128
        pltpu.sync_copy(acc_sh.at[pl.ds(base, 128)], rows.at[0])
        pltpu.sync_copy(rows.at[0], acc_ref.at[c, pl.ds(base, 128)])


def _scatter(g, sd):
    return pl.kernel(
        _scatter_body,
        out_type=jax.ShapeDtypeStruct((NC, NP, DH), jnp.float32),
        mesh=_mesh(),
        scratch_types=[
            pltpu.VMEM((HG, 2, 128), jnp.int32),
            pltpu.VMEM((2, 128, DH), jnp.float32),
            pltpu.SemaphoreType.DMA,
            pltpu.SemaphoreType.DMA,
            pltpu.VMEM_SHARED((NP, DH), jnp.float32),
        ],
    )(g, sd)


RB = 1000
RB3 = 1024


def _mm_body(x_ref, w_ref, o_ref):
    o_ref[...] = jnp.dot(x_ref[...], w_ref[...],
                         preferred_element_type=jnp.float32)


def _matmul(x, W1):
    return pl.pallas_call(
        _mm_body,
        grid=(N // RB,),
        in_specs=[pl.BlockSpec((RB, DIN), lambda i: (i, 0)),
                  pl.BlockSpec((DIN, DH), lambda i: (0, 0))],
        out_specs=pl.BlockSpec((RB, DH), lambda i: (i, 0)),
        out_shape=jax.ShapeDtypeStruct((N, DH), jnp.float32),
    )(x, W1)


def _scale_body(h_ref, s_ref, o_ref):
    o_ref[...] = h_ref[...] * s_ref[...]


def _scale(h, s_b):
    return pl.pallas_call(
        _scale_body,
        grid=(N // RB,),
        in_specs=[pl.BlockSpec((RB, DH), lambda i: (i, 0)),
                  pl.BlockSpec((RB, DH), lambda i: (i, 0))],
        out_specs=pl.BlockSpec((RB, DH), lambda i: (i, 0)),
        out_shape=jax.ShapeDtypeStruct((N, DH), jnp.float32),
    )(h, s_b)


def _layer2_body(a_ref, g1_ref, s_ref, b1_ref, w2_ref, o_ref):
    s = s_ref[...]
    t = (a_ref[0] + a_ref[1] + g1_ref[...]) * s + b1_ref[...]
    t = jnp.maximum(t, jnp.float32(0.0))
    o_ref[...] = jnp.dot(t, w2_ref[...], preferred_element_type=jnp.float32) * s


def _layer2(acc, g1, s_b, b1r, W2):
    blk = pl.BlockSpec((RB, DH), lambda i: (i, 0))
    return pl.pallas_call(
        _layer2_body,
        grid=(N // RB,),
        in_specs=[pl.BlockSpec((NC, RB, DH), lambda i: (0, i, 0)),
                  blk, blk,
                  pl.BlockSpec((1, DH), lambda i: (0, 0)),
                  pl.BlockSpec((DH, DH), lambda i: (0, 0))],
        out_specs=blk,
        out_shape=jax.ShapeDtypeStruct((N, DH), jnp.float32),
    )(acc, g1, s_b, b1r, W2)


def _pool_body(a_ref, g2_ref, s_ref, b2_ref, batch_ref, o_ref, sums, cnts):
    i = pl.program_id(0)

    @pl.when(i == 0)
    def _():
        sums[...] = jnp.zeros((G, DH), jnp.float32)
        cnts[...] = jnp.zeros((G, DH), jnp.float32)

    h2 = (a_ref[0] + a_ref[1] + g2_ref[...]) * s_ref[...] + b2_ref[...]
    bblk = batch_ref[0]
    gids = lax.broadcasted_iota(jnp.int32, (G, 128), 0)
    for r4 in range(RB3 // 128):
        brow = bblk[r4:r4 + 1, :]
        mask = (brow == gids).astype(jnp.float32)
        sums[...] += jnp.dot(mask, h2[r4 * 128:(r4 + 1) * 128, :],
                             preferred_element_type=jnp.float32)
        cnts[...] += jnp.sum(mask, axis=1, keepdims=True)

    @pl.when(i == NP // RB3 - 1)
    def _():
        o_ref[...] = sums[...] / jnp.maximum(cnts[...], jnp.float32(1.0))


def _pool(acc, g2p, s_b, b2r, batch_p):
    blk = pl.BlockSpec((RB3, DH), lambda i: (i, 0))
    return pl.pallas_call(
        _pool_body,
        grid=(NP // RB3,),
        in_specs=[pl.BlockSpec((NC, RB3, DH), lambda i: (0, i, 0)),
                  blk, blk,
                  pl.BlockSpec((1, DH), lambda i: (0, 0)),
                  pl.BlockSpec((1, RB3 // 128, 128), lambda i: (i, 0, 0))],
        out_specs=pl.BlockSpec((G, DH), lambda i: (0, 0)),
        out_shape=jax.ShapeDtypeStruct((G, DH), jnp.float32),
        scratch_shapes=[pltpu.VMEM((G, DH), jnp.float32),
                        pltpu.VMEM((G, DH), jnp.float32)],
    )(acc, g2p, s_b, b2r, batch_p)


def kernel(x, edge_index, batch, W1, b1, W2, b2):
    npad = EP - E
    pad_src = jnp.arange(npad, dtype=jnp.int32) % N
    pad_dst = (jnp.arange(npad, dtype=jnp.int32) % (NP - N)) + N
    sd = jnp.concatenate(
        [edge_index.astype(jnp.int32), jnp.stack([pad_src, pad_dst])],
        axis=1).reshape(2, EP // 128, 128).transpose(1, 0, 2)
    batch_p = jnp.concatenate(
        [batch.astype(jnp.int32),
         jnp.full((NP - N,), 127, jnp.int32)]).reshape(NP // RB3, RB3 // 128, 128)
    b1r = b1.reshape(1, DH)
    b2r = b2.reshape(1, DH)

    h1 = _matmul(x, W1)
    s_b = _deg_s(sd)
    g1 = _scale(h1, s_b)
    acc1 = _scatter(g1, sd)
    g2 = _layer2(acc1, g1, s_b, b1r, W2)
    acc2 = _scatter(g2, sd)
    g2p = jnp.pad(g2, ((0, NP - N), (0, 0)))
    return _pool(acc2, g2p, s_b, b2r, batch_p)


# speedup vs baseline: 1.2679x; 1.0156x over previous
import functools

import jax
import jax.numpy as jnp
from jax import lax
from jax.experimental import pallas as pl
from jax.experimental.pallas import tpu as pltpu
from jax.experimental.pallas import tpu_sc as plsc

N = 10000
E = 320000
G = 64
DIN = 768
DH = 128

NC = 2
NS = 16
NW = NC * NS
LANES = 16

NP = 10240
EP = 327680
NG = 80
HG = 40
ROWS_PER_TILE = NP // NW
ROWS_PER_SUB = NP // NS

_mesh = functools.partial(
    plsc.VectorSubcoreMesh, core_axis_name="c", subcore_axis_name="s",
    num_cores=NC, num_subcores=NS)


def _rsqrt16(d):
    i = lax.bitcast_convert_type(d, jnp.int32)
    i = jnp.int32(0x5F3759DF) - lax.shift_right_arithmetic(i, 1)
    y = lax.bitcast_convert_type(i, jnp.float32)
    for _ in range(3):
        y = y * (jnp.float32(1.5) - jnp.float32(0.5) * d * y * y)
    return y


def _deg_s_body(sd_ref, s_ref, sd_v, ones_v, zeros_v, sv, s_blk, hist_sh, semh):
    c = lax.axis_index("c")
    sid = lax.axis_index("s")

    def _fill(i, _):
        zeros_v[pl.ds(i * LANES, LANES)] = jnp.zeros((LANES,), jnp.float32)
        return 0
    lax.fori_loop(0, ROWS_PER_SUB // LANES, _fill, 0)
    for k in range(8):
        ones_v[pl.ds(k * LANES, LANES)] = jnp.ones((LANES,), jnp.float32)

    pltpu.sync_copy(zeros_v, hist_sh.at[pl.ds(sid * ROWS_PER_SUB, ROWS_PER_SUB)])
    plsc.subcore_barrier()

    def _chunk(chunk_idx):
        pltpu.sync_copy(sd_ref.at[pl.ds(chunk_idx * NG, NG)], sd_v)
        for j in range(NG):
            pltpu.async_copy(ones_v, hist_sh.at[sd_v.at[j, 1]], semh, add=True)
        for j in range(NG):
            pltpu.make_async_copy(ones_v, hist_sh.at[sd_v.at[j, 1]], semh).wait()

    _chunk(sid)
    _chunk(sid + NS)
    plsc.subcore_barrier()

    base = c * (NP // NC) + sid * ROWS_PER_TILE
    pltpu.sync_copy(hist_sh.at[pl.ds(base, ROWS_PER_TILE)], sv)

    def _scal(i, _):
        d = sv[pl.ds(i * LANES, LANES)] + jnp.float32(1.0)
        sv[pl.ds(i * LANES, LANES)] = _rsqrt16(d)
        return 0
    lax.fori_loop(0, ROWS_PER_TILE // LANES, _scal, 0)

    def _bcast(c16, _):
        v = sv[pl.ds(c16 * LANES, LANES)]
        for j in range(LANES):
            row = c16 * LANES + j
            vj = jnp.full((LANES,), v[j], jnp.float32)
            for k in range(DH // LANES):
                s_blk[row, pl.ds(k * LANES, LANES)] = vj
        return 0
    lax.fori_loop(0, ROWS_PER_TILE // LANES, _bcast, 0)

    pltpu.sync_copy(s_blk, s_ref.at[pl.ds(base, ROWS_PER_TILE)])


def _deg_s(sd):
    return pl.kernel(
        _deg_s_body,
        out_type=jax.ShapeDtypeStruct((NP, DH), jnp.float32),
        mesh=_mesh(),
        scratch_types=[
            pltpu.VMEM((NG, 2, 128), jnp.int32),
            pltpu.VMEM((128,), jnp.float32),
            pltpu.VMEM((ROWS_PER_SUB,), jnp.float32),
            pltpu.VMEM((ROWS_PER_TILE,), jnp.float32),
            pltpu.VMEM((ROWS_PER_TILE, DH), jnp.float32),
            pltpu.VMEM_SHARED((NP,), jnp.float32),
            pltpu.SemaphoreType.DMA,
        ],
    )(sd)


def _scatter_body(g_ref, sd_ref, acc_ref,
                  sd_v, rows, sem0, sem1, acc_sh):
    c = lax.axis_index("c")
    sid = lax.axis_index("s")
    wid = c * NS + sid

    def _z(r, _):
        for k in range(DH // LANES):
            rows[0, r, pl.ds(k * LANES, LANES)] = jnp.zeros((LANES,), jnp.float32)
        return 0
    lax.fori_loop(0, 128, _z, 0)
    for k in range(ROWS_PER_SUB // 128):
        pltpu.sync_copy(rows.at[0],
                        acc_sh.at[pl.ds(sid * ROWS_PER_SUB + k * 128, 128)])
    plsc.subcore_barrier()

    def _gat(j, b):
        return pltpu.make_async_copy(g_ref.at[sd_v.at[j, 0]], rows.at[b], sem0 if b == 0 else sem1)

    for h in range(NG // HG):
        pltpu.sync_copy(sd_ref.at[pl.ds(wid * NG + h * HG, HG)], sd_v)
        _gat(0, 0).start()

        def _grp(i, _):
            j = i * 2
            _gat(j + 1, 1).start()
            _gat(j, 0).wait()
            pltpu.sync_copy(rows.at[0], acc_sh.at[sd_v.at[j, 1]], add=True)

            @pl.when(j + 2 < HG)
            def _():
                _gat(j + 2, 0).start()
            _gat(j + 1, 1).wait()
            pltpu.sync_copy(rows.at[1], acc_sh.at[sd_v.at[j + 1, 1]], add=True)
            return 0
        lax.fori_loop(0, HG // 2, _grp, 0)
    plsc.subcore_barrier()

    for k in range(ROWS_PER_SUB // 128):
        base = sid * ROWS_PER_SUB + k * 128
        pltpu.sync_copy(acc_sh.at[pl.ds(base, 128)], rows.at[0])
        pltpu.sync_copy(rows.at[0], acc_ref.at[c, pl.ds(base, 128)])


def _scatter(g, sd):
    return pl.kernel(
        _scatter_body,
        out_type=jax.ShapeDtypeStruct((NC, NP, DH), jnp.float32),
        mesh=_mesh(),
        scratch_types=[
            pltpu.VMEM((HG, 2, 128), jnp.int32),
            pltpu.VMEM((2, 128, DH), jnp.float32),
            pltpu.SemaphoreType.DMA,
            pltpu.SemaphoreType.DMA,
            pltpu.VMEM_SHARED((NP, DH), jnp.float32),
        ],
    )(g, sd)


RB = 2000
RB3 = 1024


def _mm_body(x_ref, w_ref, o_ref):
    o_ref[...] = jnp.dot(x_ref[...], w_ref[...],
                         preferred_element_type=jnp.float32)


def _matmul(x, W1):
    return pl.pallas_call(
        _mm_body,
        grid=(N // RB,),
        in_specs=[pl.BlockSpec((RB, DIN), lambda i: (i, 0)),
                  pl.BlockSpec((DIN, DH), lambda i: (0, 0))],
        out_specs=pl.BlockSpec((RB, DH), lambda i: (i, 0)),
        out_shape=jax.ShapeDtypeStruct((N, DH), jnp.float32),
    )(x, W1)


def _scale_body(h_ref, s_ref, o_ref):
    o_ref[...] = h_ref[...] * s_ref[...]


def _scale(h, s_b):
    return pl.pallas_call(
        _scale_body,
        grid=(N // RB,),
        in_specs=[pl.BlockSpec((RB, DH), lambda i: (i, 0)),
                  pl.BlockSpec((RB, DH), lambda i: (i, 0))],
        out_specs=pl.BlockSpec((RB, DH), lambda i: (i, 0)),
        out_shape=jax.ShapeDtypeStruct((N, DH), jnp.float32),
    )(h, s_b)


def _layer2_body(a_ref, g1_ref, s_ref, b1_ref, w2_ref, o_ref):
    s = s_ref[...]
    t = (a_ref[0] + a_ref[1] + g1_ref[...]) * s + b1_ref[...]
    t = jnp.maximum(t, jnp.float32(0.0))
    o_ref[...] = jnp.dot(t, w2_ref[...], preferred_element_type=jnp.float32) * s


def _layer2(acc, g1, s_b, b1r, W2):
    blk = pl.BlockSpec((RB, DH), lambda i: (i, 0))
    return pl.pallas_call(
        _layer2_body,
        grid=(N // RB,),
        in_specs=[pl.BlockSpec((NC, RB, DH), lambda i: (0, i, 0)),
                  blk, blk,
                  pl.BlockSpec((1, DH), lambda i: (0, 0)),
                  pl.BlockSpec((DH, DH), lambda i: (0, 0))],
        out_specs=blk,
        out_shape=jax.ShapeDtypeStruct((N, DH), jnp.float32),
    )(acc, g1, s_b, b1r, W2)


def _pool_body(a_ref, g2_ref, s_ref, b2_ref, batch_ref, o_ref, sums, cnts):
    i = pl.program_id(0)

    @pl.when(i == 0)
    def _():
        sums[...] = jnp.zeros((G, DH), jnp.float32)
        cnts[...] = jnp.zeros((G, DH), jnp.float32)

    h2 = (a_ref[0] + a_ref[1] + g2_ref[...]) * s_ref[...] + b2_ref[...]
    bblk = batch_ref[0]
    gids = lax.broadcasted_iota(jnp.int32, (G, 128), 0)
    for r4 in range(RB3 // 128):
        brow = bblk[r4:r4 + 1, :]
        mask = (brow == gids).astype(jnp.float32)
        sums[...] += jnp.dot(mask, h2[r4 * 128:(r4 + 1) * 128, :],
                             preferred_element_type=jnp.float32)
        cnts[...] += jnp.sum(mask, axis=1, keepdims=True)

    @pl.when(i == NP // RB3 - 1)
    def _():
        o_ref[...] = sums[...] / jnp.maximum(cnts[...], jnp.float32(1.0))


def _pool(acc, g2p, s_b, b2r, batch_p):
    blk = pl.BlockSpec((RB3, DH), lambda i: (i, 0))
    return pl.pallas_call(
        _pool_body,
        grid=(NP // RB3,),
        in_specs=[pl.BlockSpec((NC, RB3, DH), lambda i: (0, i, 0)),
                  blk, blk,
                  pl.BlockSpec((1, DH), lambda i: (0, 0)),
                  pl.BlockSpec((1, RB3 // 128, 128), lambda i: (i, 0, 0))],
        out_specs=pl.BlockSpec((G, DH), lambda i: (0, 0)),
        out_shape=jax.ShapeDtypeStruct((G, DH), jnp.float32),
        scratch_shapes=[pltpu.VMEM((G, DH), jnp.float32),
                        pltpu.VMEM((G, DH), jnp.float32)],
    )(acc, g2p, s_b, b2r, batch_p)


def kernel(x, edge_index, batch, W1, b1, W2, b2):
    npad = EP - E
    pad_src = jnp.arange(npad, dtype=jnp.int32) % N
    pad_dst = (jnp.arange(npad, dtype=jnp.int32) % (NP - N)) + N
    sd_real = edge_index.astype(jnp.int32).reshape(2, E // 128, 128).transpose(1, 0, 2)
    sd_pad = jnp.stack([pad_src.reshape(npad // 128, 128),
                        pad_dst.reshape(npad // 128, 128)], axis=1)
    sd = jnp.concatenate([sd_real, sd_pad], axis=0)
    batch_p = jnp.concatenate(
        [batch.astype(jnp.int32),
         jnp.full((NP - N,), 127, jnp.int32)]).reshape(NP // RB3, RB3 // 128, 128)
    b1r = b1.reshape(1, DH)
    b2r = b2.reshape(1, DH)

    h1 = _matmul(x, W1)
    s_b = _deg_s(sd)
    g1 = _scale(h1, s_b)
    acc1 = _scatter(g1, sd)
    g2 = _layer2(acc1, g1, s_b, b1r, W2)
    acc2 = _scatter(g2, sd)
    g2p = jnp.pad(g2, ((0, NP - N), (0, 0)))
    return _pool(acc2, g2p, s_b, b2r, batch_p)


# speedup vs baseline: 1.2866x; 1.0148x over previous
import functools

import jax
import jax.numpy as jnp
from jax import lax
from jax.experimental import pallas as pl
from jax.experimental.pallas import tpu as pltpu
from jax.experimental.pallas import tpu_sc as plsc

N = 10000
E = 320000
G = 64
DIN = 768
DH = 128

NC = 2
NS = 16
NW = NC * NS
LANES = 16

NP = 10240
EP = 327680
NG = 80
HG = 40
ROWS_PER_TILE = NP // NW
ROWS_PER_SUB = NP // NS

_mesh = functools.partial(
    plsc.VectorSubcoreMesh, core_axis_name="c", subcore_axis_name="s",
    num_cores=NC, num_subcores=NS)


def _rsqrt16(d):
    i = lax.bitcast_convert_type(d, jnp.int32)
    i = jnp.int32(0x5F3759DF) - lax.shift_right_arithmetic(i, 1)
    y = lax.bitcast_convert_type(i, jnp.float32)
    for _ in range(3):
        y = y * (jnp.float32(1.5) - jnp.float32(0.5) * d * y * y)
    return y


def _deg_s_body(sd_ref, s_ref, sd_v, sd_v2, ones_v, zeros_v, sv, s_blk,
                hist_sh, semh, semh2):
    c = lax.axis_index("c")
    sid = lax.axis_index("s")

    def _fill(i, _):
        zeros_v[pl.ds(i * LANES, LANES)] = jnp.zeros((LANES,), jnp.float32)
        return 0
    lax.fori_loop(0, ROWS_PER_SUB // LANES, _fill, 0)
    for k in range(8):
        ones_v[pl.ds(k * LANES, LANES)] = jnp.ones((LANES,), jnp.float32)

    pltpu.sync_copy(zeros_v, hist_sh.at[pl.ds(sid * ROWS_PER_SUB, ROWS_PER_SUB)])
    plsc.subcore_barrier()

    pltpu.sync_copy(sd_ref.at[pl.ds(sid * NG, NG)], sd_v)
    ld2 = pltpu.make_async_copy(
        sd_ref.at[pl.ds((sid + NS) * NG, NG)], sd_v2, semh2)
    ld2.start()
    for buf in (sd_v, sd_v2):
        if buf is sd_v2:
            ld2.wait()
        for j in range(NG):
            pltpu.async_copy(ones_v, hist_sh.at[buf.at[j, 1]], semh, add=True)
        for j in range(NG):
            pltpu.make_async_copy(ones_v, hist_sh.at[buf.at[j, 1]], semh).wait()
    plsc.subcore_barrier()

    base = c * (NP // NC) + sid * ROWS_PER_TILE
    pltpu.sync_copy(hist_sh.at[pl.ds(base, ROWS_PER_TILE)], sv)

    def _scal(i, _):
        d = sv[pl.ds(i * LANES, LANES)] + jnp.float32(1.0)
        sv[pl.ds(i * LANES, LANES)] = _rsqrt16(d)
        return 0
    lax.fori_loop(0, ROWS_PER_TILE // LANES, _scal, 0)

    def _bcast(c16, _):
        v = sv[pl.ds(c16 * LANES, LANES)]
        for j in range(LANES):
            row = c16 * LANES + j
            vj = jnp.full((LANES,), v[j], jnp.float32)
            for k in range(DH // LANES):
                s_blk[row, pl.ds(k * LANES, LANES)] = vj
        return 0
    lax.fori_loop(0, ROWS_PER_TILE // LANES, _bcast, 0)

    pltpu.sync_copy(s_blk, s_ref.at[pl.ds(base, ROWS_PER_TILE)])


def _deg_s(sd):
    return pl.kernel(
        _deg_s_body,
        out_type=jax.ShapeDtypeStruct((NP, DH), jnp.float32),
        mesh=_mesh(),
        scratch_types=[
            pltpu.VMEM((NG, 2, 128), jnp.int32),
            pltpu.VMEM((NG, 2, 128), jnp.int32),
            pltpu.VMEM((128,), jnp.float32),
            pltpu.VMEM((ROWS_PER_SUB,), jnp.float32),
            pltpu.VMEM((ROWS_PER_TILE,), jnp.float32),
            pltpu.VMEM((ROWS_PER_TILE, DH), jnp.float32),
            pltpu.VMEM_SHARED((NP,), jnp.float32),
            pltpu.SemaphoreType.DMA,
            pltpu.SemaphoreType.DMA,
        ],
    )(sd)


def _scatter_body(g_ref, sd_ref, acc_ref,
                  sd_v, rows, sem0, sem1, acc_sh):
    c = lax.axis_index("c")
    sid = lax.axis_index("s")
    wid = c * NS + sid

    def _z(r, _):
        for k in range(DH // LANES):
            rows[0, r, pl.ds(k * LANES, LANES)] = jnp.zeros((LANES,), jnp.float32)
        return 0
    lax.fori_loop(0, 128, _z, 0)
    for k in range(ROWS_PER_SUB // 128):
        pltpu.sync_copy(rows.at[0],
                        acc_sh.at[pl.ds(sid * ROWS_PER_SUB + k * 128, 128)])
    plsc.subcore_barrier()

    def _gat(j, b):
        return pltpu.make_async_copy(g_ref.at[sd_v.at[j, 0]], rows.at[b], sem0 if b == 0 else sem1)

    for h in range(NG // HG):
        pltpu.sync_copy(sd_ref.at[pl.ds(wid * NG + h * HG, HG)], sd_v)
        _gat(0, 0).start()

        def _grp(i, _):
            j = i * 2
            _gat(j + 1, 1).start()
            _gat(j, 0).wait()
            pltpu.sync_copy(rows.at[0], acc_sh.at[sd_v.at[j, 1]], add=True)

            @pl.when(j + 2 < HG)
            def _():
                _gat(j + 2, 0).start()
            _gat(j + 1, 1).wait()
            pltpu.sync_copy(rows.at[1], acc_sh.at[sd_v.at[j + 1, 1]], add=True)
            return 0
        lax.fori_loop(0, HG // 2, _grp, 0)
    plsc.subcore_barrier()

    for k in range(ROWS_PER_SUB // 128):
        base = sid * ROWS_PER_SUB + k * 128
        pltpu.sync_copy(acc_sh.at[pl.ds(base, 128)], rows.at[0])
        pltpu.sync_copy(rows.at[0], acc_ref.at[c, pl.ds(base, 128)])


def _scatter(g, sd):
    return pl.kernel(
        _scatter_body,
        out_type=jax.ShapeDtypeStruct((NC, NP, DH), jnp.float32),
        mesh=_mesh(),
        scratch_types=[
            pltpu.VMEM((HG, 2, 128), jnp.int32),
            pltpu.VMEM((2, 128, DH), jnp.float32),
            pltpu.SemaphoreType.DMA,
            pltpu.SemaphoreType.DMA,
            pltpu.VMEM_SHARED((NP, DH), jnp.float32),
        ],
    )(g, sd)


RB = 2000
RB3 = 1024


def _mm_body(x_ref, w_ref, o_ref):
    o_ref[...] = jnp.dot(x_ref[...], w_ref[...],
                         preferred_element_type=jnp.float32)


def _matmul(x, W1):
    return pl.pallas_call(
        _mm_body,
        grid=(N // RB,),
        in_specs=[pl.BlockSpec((RB, DIN), lambda i: (i, 0)),
                  pl.BlockSpec((DIN, DH), lambda i: (0, 0))],
        out_specs=pl.BlockSpec((RB, DH), lambda i: (i, 0)),
        out_shape=jax.ShapeDtypeStruct((N, DH), jnp.float32),
    )(x, W1)


def _scale_body(h_ref, s_ref, o_ref):
    o_ref[...] = h_ref[...] * s_ref[...]


def _scale(h, s_b):
    return pl.pallas_call(
        _scale_body,
        grid=(N // RB,),
        in_specs=[pl.BlockSpec((RB, DH), lambda i: (i, 0)),
                  pl.BlockSpec((RB, DH), lambda i: (i, 0))],
        out_specs=pl.BlockSpec((RB, DH), lambda i: (i, 0)),
        out_shape=jax.ShapeDtypeStruct((N, DH), jnp.float32),
    )(h, s_b)


def _layer2_body(a_ref, g1_ref, s_ref, b1_ref, w2_ref, o_ref):
    s = s_ref[...]
    t = (a_ref[0] + a_ref[1] + g1_ref[...]) * s + b1_ref[...]
    t = jnp.maximum(t, jnp.float32(0.0))
    o_ref[...] = jnp.dot(t, w2_ref[...], preferred_element_type=jnp.float32) * s


def _layer2(acc, g1, s_b, b1r, W2):
    blk = pl.BlockSpec((RB, DH), lambda i: (i, 0))
    return pl.pallas_call(
        _layer2_body,
        grid=(N // RB,),
        in_specs=[pl.BlockSpec((NC, RB, DH), lambda i: (0, i, 0)),
                  blk, blk,
                  pl.BlockSpec((1, DH), lambda i: (0, 0)),
                  pl.BlockSpec((DH, DH), lambda i: (0, 0))],
        out_specs=blk,
        out_shape=jax.ShapeDtypeStruct((N, DH), jnp.float32),
    )(acc, g1, s_b, b1r, W2)


def _pool_body(a_ref, g2_ref, s_ref, b2_ref, batch_ref, o_ref, sums, cnts):
    i = pl.program_id(0)

    @pl.when(i == 0)
    def _():
        sums[...] = jnp.zeros((G, DH), jnp.float32)
        cnts[...] = jnp.zeros((G, DH), jnp.float32)

    h2 = (a_ref[0] + a_ref[1] + g2_ref[...]) * s_ref[...] + b2_ref[...]
    row = i * RB3 + lax.broadcasted_iota(jnp.int32, (RB3, 1), 0)
    h2 = jnp.where(row < N, h2, jnp.float32(0.0))
    bblk = batch_ref[0]
    gids = lax.broadcasted_iota(jnp.int32, (G, 128), 0)
    for r4 in range(RB3 // 128):
        brow = bblk[r4:r4 + 1, :]
        mask = (brow == gids).astype(jnp.float32)
        sums[...] += jnp.dot(mask, h2[r4 * 128:(r4 + 1) * 128, :],
                             preferred_element_type=jnp.float32)
        cnts[...] += jnp.sum(mask, axis=1, keepdims=True)

    @pl.when(i == NP // RB3 - 1)
    def _():
        o_ref[...] = sums[...] / jnp.maximum(cnts[...], jnp.float32(1.0))


def _pool(acc, g2, s_b, b2r, batch_p):
    blk = pl.BlockSpec((RB3, DH), lambda i: (i, 0))
    return pl.pallas_call(
        _pool_body,
        grid=(NP // RB3,),
        in_specs=[pl.BlockSpec((NC, RB3, DH), lambda i: (0, i, 0)),
                  blk, blk,
                  pl.BlockSpec((1, DH), lambda i: (0, 0)),
                  pl.BlockSpec((1, RB3 // 128, 128), lambda i: (i, 0, 0))],
        out_specs=pl.BlockSpec((G, DH), lambda i: (0, 0)),
        out_shape=jax.ShapeDtypeStruct((G, DH), jnp.float32),
        scratch_shapes=[pltpu.VMEM((G, DH), jnp.float32),
                        pltpu.VMEM((G, DH), jnp.float32)],
    )(acc, g2, s_b, b2r, batch_p)


def kernel(x, edge_index, batch, W1, b1, W2, b2):
    npad = EP - E
    pad_src = jnp.arange(npad, dtype=jnp.int32) % N
    pad_dst = (jnp.arange(npad, dtype=jnp.int32) % (NP - N)) + N
    sd_real = edge_index.astype(jnp.int32).reshape(2, E // 128, 128).transpose(1, 0, 2)
    sd_pad = jnp.stack([pad_src.reshape(npad // 128, 128),
                        pad_dst.reshape(npad // 128, 128)], axis=1)
    sd = jnp.concatenate([sd_real, sd_pad], axis=0)
    batch_p = jnp.concatenate(
        [batch.astype(jnp.int32),
         jnp.full((NP - N,), 127, jnp.int32)]).reshape(NP // RB3, RB3 // 128, 128)
    b1r = b1.reshape(1, DH)
    b2r = b2.reshape(1, DH)

    h1 = _matmul(x, W1)
    s_b = _deg_s(sd)
    g1 = _scale(h1, s_b)
    acc1 = _scatter(g1, sd)
    g2 = _layer2(acc1, g1, s_b, b1r, W2)
    acc2 = _scatter(g2, sd)
    return _pool(acc2, g2, s_b, b2r, batch_p)
